# Initial kernel scaffold; baseline (speedup 1.0000x reference)
import jax
import jax.numpy as jnp
from jax.experimental import pallas as pl


def kernel(x, edge_index, W1, b1, W2, b2):
    raise NotImplementedError("write your pallas kernel here")


# speedup vs baseline: 6.4557x; 6.4557x over previous
import jax
import jax.numpy as jnp
from jax import lax
from jax.experimental import pallas as pl
from jax.experimental.pallas import tpu as pltpu
from jax.experimental.pallas import tpu_sc as plsc

N = 10000
NP = 10240
E = 320000
D = 128
C = 16
NC, NS, L = 2, 16, 16
NW = NC * NS
ET = NP
EP = NW * ET
K = 128
NCH = ET // K
ROWS_PT = NP // NS
HW = 16

_MESH = plsc.VectorSubcoreMesh(
    core_axis_name="c", subcore_axis_name="s", num_cores=NC, num_subcores=NS)


def _fill_rows(buf, value, rows, width):
  vec = jnp.full((L,), value, jnp.float32)

  def body(i, carry):
    for k in range(width // L):
      buf[i, pl.ds(k * L, L)] = vec
    return carry

  lax.fori_loop(0, rows, body, 0)


def _hist_body(hidx_hbm, hist_hbm, idx_v, ones_v, hist_sh, s0, s1, s2, s3):
  cidx = lax.axis_index("c")
  sidx = lax.axis_index("s")
  wid = cidx * NS + sidx
  sems = [s0, s1, s2, s3]
  ndepth = len(sems)

  pltpu.sync_copy(hidx_hbm.at[wid], idx_v)

  _fill_rows(ones_v, 0.0, K, HW)
  zrows = 2 * NP // NS
  zbase = sidx * zrows
  for r in range(zrows // K):
    pltpu.sync_copy(ones_v, hist_sh.at[pl.ds(zbase + r * K, K)])
  _fill_rows(ones_v, 1.0, K, HW)
  plsc.subcore_barrier()

  if True:
    def loop(i, carry):
      for b in range(ndepth):
        jj = i * ndepth + b

        @pl.when(i > 0)
        def _(b=b):
          pltpu.make_async_copy(ones_v, hist_sh.at[idx_v.at[0]],
                                sems[b]).wait()

        pltpu.async_copy(ones_v, hist_sh.at[idx_v.at[jj]], sems[b], add=True)
      return carry

    lax.fori_loop(0, (2 * NCH) // ndepth, loop, 0)
    for b in range(ndepth):
      pltpu.make_async_copy(ones_v, hist_sh.at[idx_v.at[0]], sems[b]).wait()
  plsc.subcore_barrier()

  pltpu.sync_copy(hist_sh.at[pl.ds(zbase, zrows)],
                  hist_hbm.at[cidx, pl.ds(zbase, zrows)])


_hist = pl.kernel(
    _hist_body,
    out_type=jax.ShapeDtypeStruct((NC, 2 * NP, HW), jnp.float32),
    mesh=_MESH,
    compiler_params=pltpu.CompilerParams(use_tc_tiling_on_sc=False),
    scratch_types=[
        pltpu.VMEM((2 * NCH, K), jnp.int32),
        pltpu.VMEM((K, HW), jnp.float32),
        pltpu.VMEM_SHARED((2 * NP, HW), jnp.float32),
        pltpu.SemaphoreType.DMA,
        pltpu.SemaphoreType.DMA,
        pltpu.SemaphoreType.DMA,
        pltpu.SemaphoreType.DMA,
    ],
)


NHALF = 2
NCH2 = NCH // NHALF


def _make_agg(W):
  def body(table_hbm, src_hbm, dst_hbm, out_hbm,
           idx_s, idx_d, buf0, buf1, accum_sh, g0, g1, sa0, sa1):
    cidx = lax.axis_index("c")
    sidx = lax.axis_index("s")
    wid = cidx * NS + sidx
    bufs = [buf0, buf1]
    gsem = [g0, g1]
    ssem = [sa0, sa1]

    _fill_rows(buf0, 0.0, K, W)
    for r in range(ROWS_PT // K):
      pltpu.sync_copy(buf0, accum_sh.at[pl.ds(sidx * ROWS_PT + r * K, K)])
    plsc.subcore_barrier()

    for h in range(NHALF):
      pltpu.sync_copy(src_hbm.at[wid, pl.ds(h * NCH2, NCH2)], idx_s)
      pltpu.sync_copy(dst_hbm.at[wid, pl.ds(h * NCH2, NCH2)], idx_d)

      pltpu.async_copy(table_hbm.at[idx_s.at[0]], buf0, g0)

      def loop(i, carry):
        for b in range(2):
          jj = 2 * i + b
          ob = 1 - b
          pltpu.make_async_copy(table_hbm.at[idx_s.at[0]], bufs[b],
                                gsem[b]).wait()

          @pl.when(jj + 1 < NCH2)
          def _(b=b, ob=ob, jj=jj):
            @pl.when(jj >= 1)
            def _():
              pltpu.make_async_copy(bufs[ob], accum_sh.at[idx_d.at[0]],
                                    ssem[ob]).wait()
            pltpu.async_copy(table_hbm.at[idx_s.at[jj + 1]], bufs[ob],
                             gsem[ob])

          pltpu.async_copy(bufs[b], accum_sh.at[idx_d.at[jj]], ssem[b],
                           add=True)
        return carry

      lax.fori_loop(0, NCH2 // 2, loop, 0)
      pltpu.make_async_copy(bufs[0], accum_sh.at[idx_d.at[0]], ssem[0]).wait()
      pltpu.make_async_copy(bufs[1], accum_sh.at[idx_d.at[0]], ssem[1]).wait()

    plsc.subcore_barrier()

    pltpu.sync_copy(accum_sh.at[pl.ds(sidx * ROWS_PT, ROWS_PT)],
                    out_hbm.at[cidx, pl.ds(sidx * ROWS_PT, ROWS_PT)])

  return pl.kernel(
      body,
      out_type=jax.ShapeDtypeStruct((NC, NP, W), jnp.float32),
      mesh=_MESH,
      compiler_params=pltpu.CompilerParams(use_tc_tiling_on_sc=False),
      scratch_types=[
          pltpu.VMEM((NCH2, K), jnp.int32),
          pltpu.VMEM((NCH2, K), jnp.int32),
          pltpu.VMEM((K, W), jnp.float32),
          pltpu.VMEM((K, W), jnp.float32),
          pltpu.VMEM_SHARED((NP, W), jnp.float32),
          pltpu.SemaphoreType.DMA,
          pltpu.SemaphoreType.DMA,
          pltpu.SemaphoreType.DMA,
          pltpu.SemaphoreType.DMA,
      ],
  )


_agg128 = _make_agg(D)
_agg16 = _make_agg(C)


def _norm(a_ref, b_ref):
  deg = a_ref[:, 0:1] + b_ref[:, 0:1]
  return jnp.where(deg > 0, lax.rsqrt(jnp.maximum(deg, 1.0)), 0.0)


def _tc_scale_body(ho0, ho1, x_ref, xs_ref):
  xs_ref[...] = x_ref[...] * _norm(ho0, ho1)


_tc_scale = pl.pallas_call(
    _tc_scale_body,
    grid=(NP // K,),
    in_specs=[
        pl.BlockSpec((K, HW), lambda i: (i, 0)),
        pl.BlockSpec((K, HW), lambda i: (i, 0)),
        pl.BlockSpec((K, D), lambda i: (i, 0)),
    ],
    out_specs=pl.BlockSpec((K, D), lambda i: (i, 0)),
    out_shape=jax.ShapeDtypeStruct((NP, D), jnp.float32),
)


def _tc_dense_body(p0, p1, hi0, hi1, ho0, ho1, w1, bb1, w2, bb2, out):
  a = p0[...] + p1[...]
  nd = _norm(hi0, hi1)
  ns = _norm(ho0, ho1)
  z = jnp.dot(a, w1[...], preferred_element_type=jnp.float32) * nd + bb1[...]
  h = jnp.maximum(z, 0.0)
  out[...] = jnp.dot(h, w2[...], preferred_element_type=jnp.float32) * ns


_tc_dense = pl.pallas_call(
    _tc_dense_body,
    grid=(NP // K,),
    in_specs=[
        pl.BlockSpec((K, D), lambda i: (i, 0)),
        pl.BlockSpec((K, D), lambda i: (i, 0)),
        pl.BlockSpec((K, HW), lambda i: (i, 0)),
        pl.BlockSpec((K, HW), lambda i: (i, 0)),
        pl.BlockSpec((K, HW), lambda i: (i, 0)),
        pl.BlockSpec((K, HW), lambda i: (i, 0)),
        pl.BlockSpec((D, D), lambda i: (0, 0)),
        pl.BlockSpec((1, D), lambda i: (0, 0)),
        pl.BlockSpec((D, C), lambda i: (0, 0)),
        pl.BlockSpec((1, C), lambda i: (0, 0)),
    ],
    out_specs=pl.BlockSpec((K, C), lambda i: (i, 0)),
    out_shape=jax.ShapeDtypeStruct((NP, C), jnp.float32),
)


def _tc_read_body(q0, q1, hi0, hi1, bb2, out):
  i = pl.program_id(0)
  nd = _norm(hi0, hi1)
  pre = jnp.maximum((q0[...] + q1[...]) * nd + bb2[...], 0.0)
  rows = lax.broadcasted_iota(jnp.int32, (K, 1), 0) + i * K
  pre = jnp.where(rows < N, pre, 0.0)
  s = jnp.sum(pre, axis=0, keepdims=True)

  @pl.when(i == 0)
  def _():
    out[...] = jnp.zeros_like(out)

  out[...] += s

  @pl.when(i == NP // K - 1)
  def _():
    out[...] = out[...] * (1.0 / N)


_tc_read = pl.pallas_call(
    _tc_read_body,
    grid=(NP // K,),
    in_specs=[
        pl.BlockSpec((K, C), lambda i: (i, 0)),
        pl.BlockSpec((K, C), lambda i: (i, 0)),
        pl.BlockSpec((K, HW), lambda i: (i, 0)),
        pl.BlockSpec((K, HW), lambda i: (i, 0)),
        pl.BlockSpec((1, C), lambda i: (0, 0)),
    ],
    out_specs=pl.BlockSpec((1, C), lambda i: (0, 0)),
    out_shape=jax.ShapeDtypeStruct((1, C), jnp.float32),
)


def kernel(x, edge_index, W1, b1, W2, b2):
  src = edge_index[0].astype(jnp.int32)
  dst = edge_index[1].astype(jnp.int32)
  padv = jnp.full((EP - E,), N, jnp.int32)
  srcp = jnp.concatenate([src, padv])
  dstp = jnp.concatenate([dst, padv])
  src3 = srcp.reshape(NW, NCH, K)
  dst3 = dstp.reshape(NW, NCH, K)
  hidx = jnp.concatenate([srcp, dstp + NP]).reshape(NW, 2 * NCH, K)
  x_pad = jnp.pad(x, ((0, NP - N), (0, 0)))

  hist = _hist(hidx)
  ho0, ho1 = hist[0, :NP, :], hist[1, :NP, :]
  hi0, hi1 = hist[0, NP:, :], hist[1, NP:, :]

  xs = _tc_scale(ho0, ho1, x_pad)
  parts = _agg128(xs, src3, dst3)
  gs = _tc_dense(parts[0], parts[1], hi0, hi1, ho0, ho1,
                 W1, b1.reshape(1, D), W2, b2.reshape(1, C))
  parts2 = _agg16(gs, src3, dst3)
  return _tc_read(parts2[0], parts2[1], hi0, hi1, b2.reshape(1, C))


# speedup vs baseline: 13.5068x; 2.0922x over previous
import jax
import jax.numpy as jnp
from jax import lax
from jax.experimental import pallas as pl
from jax.experimental.pallas import tpu as pltpu
from jax.experimental.pallas import tpu_sc as plsc

N = 10000
NP = 10240
E = 320000
D = 128
C = 16
NC, NS, L = 2, 16, 16
NW = NC * NS
ET = NP
EP = NW * ET
K = 128
NCH = ET // K
ROWS_PT = NP // NS
HW = 16

_MESH = plsc.VectorSubcoreMesh(
    core_axis_name="c", subcore_axis_name="s", num_cores=NC, num_subcores=NS)


def _fill_rows(buf, value, rows, width):
  vec = jnp.full((L,), value, jnp.float32)

  def body(i, carry):
    for k in range(width // L):
      buf[i, pl.ds(k * L, L)] = vec
    return carry

  lax.fori_loop(0, rows, body, 0)


def _hist_body(hidx_hbm, hist_hbm, idx_v, ones_v, hist_sh, s0, s1, s2, s3):
  cidx = lax.axis_index("c")
  sidx = lax.axis_index("s")
  wid = cidx * NS + sidx
  sems = [s0, s1, s2, s3]
  ndepth = len(sems)

  pltpu.sync_copy(hidx_hbm.at[wid], idx_v)

  _fill_rows(ones_v, 0.0, K, HW)
  zrows = 2 * NP // NS
  zbase = sidx * zrows
  for r in range(zrows // K):
    pltpu.sync_copy(ones_v, hist_sh.at[pl.ds(zbase + r * K, K)])
  _fill_rows(ones_v, 1.0, K, HW)
  plsc.subcore_barrier()

  if True:
    def loop(i, carry):
      for b in range(ndepth):
        jj = i * ndepth + b

        @pl.when(i > 0)
        def _(b=b):
          pltpu.make_async_copy(ones_v, hist_sh.at[idx_v.at[0]],
                                sems[b]).wait()

        pltpu.async_copy(ones_v, hist_sh.at[idx_v.at[jj]], sems[b], add=True)
      return carry

    lax.fori_loop(0, (2 * NCH) // ndepth, loop, 0)
    for b in range(ndepth):
      pltpu.make_async_copy(ones_v, hist_sh.at[idx_v.at[0]], sems[b]).wait()
  plsc.subcore_barrier()

  pltpu.sync_copy(hist_sh.at[pl.ds(zbase, zrows)],
                  hist_hbm.at[cidx, pl.ds(zbase, zrows)])


_hist = pl.kernel(
    _hist_body,
    out_type=jax.ShapeDtypeStruct((NC, 2 * NP, HW), jnp.float32),
    mesh=_MESH,
    compiler_params=pltpu.CompilerParams(use_tc_tiling_on_sc=False),
    scratch_types=[
        pltpu.VMEM((2 * NCH, K), jnp.int32),
        pltpu.VMEM((K, HW), jnp.float32),
        pltpu.VMEM_SHARED((2 * NP, HW), jnp.float32),
        pltpu.SemaphoreType.DMA,
        pltpu.SemaphoreType.DMA,
        pltpu.SemaphoreType.DMA,
        pltpu.SemaphoreType.DMA,
    ],
)


NHALF = 2
NCH2 = NCH // NHALF


def _make_agg(W):
  def body(table_hbm, src_hbm, dst_hbm, out_hbm,
           idx_s, idx_d, buf0, buf1, accum_sh, g0, g1, sa0, sa1):
    cidx = lax.axis_index("c")
    sidx = lax.axis_index("s")
    wid = cidx * NS + sidx
    bufs = [buf0, buf1]
    gsem = [g0, g1]
    ssem = [sa0, sa1]

    _fill_rows(buf0, 0.0, K, W)
    for r in range(ROWS_PT // K):
      pltpu.sync_copy(buf0, accum_sh.at[pl.ds(sidx * ROWS_PT + r * K, K)])
    plsc.subcore_barrier()

    for h in range(NHALF):
      pltpu.sync_copy(src_hbm.at[wid, pl.ds(h * NCH2, NCH2)], idx_s)
      pltpu.sync_copy(dst_hbm.at[wid, pl.ds(h * NCH2, NCH2)], idx_d)

      pltpu.async_copy(table_hbm.at[idx_s.at[0]], buf0, g0)

      def loop(i, carry):
        for b in range(2):
          jj = 2 * i + b
          ob = 1 - b
          pltpu.make_async_copy(table_hbm.at[idx_s.at[0]], bufs[b],
                                gsem[b]).wait()

          @pl.when(jj + 1 < NCH2)
          def _(b=b, ob=ob, jj=jj):
            @pl.when(jj >= 1)
            def _():
              pltpu.make_async_copy(bufs[ob], accum_sh.at[idx_d.at[0]],
                                    ssem[ob]).wait()
            pltpu.async_copy(table_hbm.at[idx_s.at[jj + 1]], bufs[ob],
                             gsem[ob])

          pltpu.async_copy(bufs[b], accum_sh.at[idx_d.at[jj]], ssem[b],
                           add=True)
        return carry

      lax.fori_loop(0, NCH2 // 2, loop, 0)
      pltpu.make_async_copy(bufs[0], accum_sh.at[idx_d.at[0]], ssem[0]).wait()
      pltpu.make_async_copy(bufs[1], accum_sh.at[idx_d.at[0]], ssem[1]).wait()

    plsc.subcore_barrier()

    pltpu.sync_copy(accum_sh.at[pl.ds(sidx * ROWS_PT, ROWS_PT)],
                    out_hbm.at[cidx, pl.ds(sidx * ROWS_PT, ROWS_PT)])

  return pl.kernel(
      body,
      out_type=jax.ShapeDtypeStruct((NC, NP, W), jnp.float32),
      mesh=_MESH,
      compiler_params=pltpu.CompilerParams(use_tc_tiling_on_sc=False),
      scratch_types=[
          pltpu.VMEM((NCH2, K), jnp.int32),
          pltpu.VMEM((NCH2, K), jnp.int32),
          pltpu.VMEM((K, W), jnp.float32),
          pltpu.VMEM((K, W), jnp.float32),
          pltpu.VMEM_SHARED((NP, W), jnp.float32),
          pltpu.SemaphoreType.DMA,
          pltpu.SemaphoreType.DMA,
          pltpu.SemaphoreType.DMA,
          pltpu.SemaphoreType.DMA,
      ],
  )


_agg128 = _make_agg(D)
_agg16 = _make_agg(C)


def _norm(a_ref, b_ref):
  deg = a_ref[:, 0:1] + b_ref[:, 0:1]
  return jnp.where(deg > 0, lax.rsqrt(jnp.maximum(deg, 1.0)), 0.0)


def _tc_scale_body(ho0, ho1, x_ref, xs_ref):
  xs_ref[...] = x_ref[...] * _norm(ho0, ho1)


_tc_scale = pl.pallas_call(
    _tc_scale_body,
    grid=(NP // K,),
    in_specs=[
        pl.BlockSpec((K, HW), lambda i: (i, 0)),
        pl.BlockSpec((K, HW), lambda i: (i, 0)),
        pl.BlockSpec((K, D), lambda i: (i, 0)),
    ],
    out_specs=pl.BlockSpec((K, D), lambda i: (i, 0)),
    out_shape=jax.ShapeDtypeStruct((NP, D), jnp.float32),
)


def _tc_dense_body(p0, p1, hi0, hi1, ho0, ho1, w1, bb1, w2, bb2, out):
  a = p0[...] + p1[...]
  nd = _norm(hi0, hi1)
  ns = _norm(ho0, ho1)
  z = jnp.dot(a, w1[...], preferred_element_type=jnp.float32) * nd + bb1[...]
  h = jnp.maximum(z, 0.0)
  out[...] = jnp.dot(h, w2[...], preferred_element_type=jnp.float32) * ns


_tc_dense = pl.pallas_call(
    _tc_dense_body,
    grid=(NP // K,),
    in_specs=[
        pl.BlockSpec((K, D), lambda i: (i, 0)),
        pl.BlockSpec((K, D), lambda i: (i, 0)),
        pl.BlockSpec((K, HW), lambda i: (i, 0)),
        pl.BlockSpec((K, HW), lambda i: (i, 0)),
        pl.BlockSpec((K, HW), lambda i: (i, 0)),
        pl.BlockSpec((K, HW), lambda i: (i, 0)),
        pl.BlockSpec((D, D), lambda i: (0, 0)),
        pl.BlockSpec((1, D), lambda i: (0, 0)),
        pl.BlockSpec((D, C), lambda i: (0, 0)),
        pl.BlockSpec((1, C), lambda i: (0, 0)),
    ],
    out_specs=pl.BlockSpec((K, C), lambda i: (i, 0)),
    out_shape=jax.ShapeDtypeStruct((NP, C), jnp.float32),
)


def _tc_read_body(q0, q1, hi0, hi1, bb2, out):
  i = pl.program_id(0)
  nd = _norm(hi0, hi1)
  pre = jnp.maximum((q0[...] + q1[...]) * nd + bb2[...], 0.0)
  rows = lax.broadcasted_iota(jnp.int32, (K, 1), 0) + i * K
  pre = jnp.where(rows < N, pre, 0.0)
  s = jnp.sum(pre, axis=0, keepdims=True)

  @pl.when(i == 0)
  def _():
    out[...] = jnp.zeros_like(out)

  out[...] += s

  @pl.when(i == NP // K - 1)
  def _():
    out[...] = out[...] * (1.0 / N)


_tc_read = pl.pallas_call(
    _tc_read_body,
    grid=(NP // K,),
    in_specs=[
        pl.BlockSpec((K, C), lambda i: (i, 0)),
        pl.BlockSpec((K, C), lambda i: (i, 0)),
        pl.BlockSpec((K, HW), lambda i: (i, 0)),
        pl.BlockSpec((K, HW), lambda i: (i, 0)),
        pl.BlockSpec((1, C), lambda i: (0, 0)),
    ],
    out_specs=pl.BlockSpec((1, C), lambda i: (0, 0)),
    out_shape=jax.ShapeDtypeStruct((1, C), jnp.float32),
)


def kernel(x, edge_index, W1, b1, W2, b2):
  src = edge_index[0].astype(jnp.int32)
  dst = edge_index[1].astype(jnp.int32)
  padv = N + (jnp.arange(EP - E, dtype=jnp.int32) % (NP - N))
  srcp = jnp.concatenate([src, padv])
  dstp = jnp.concatenate([dst, padv])
  src3 = srcp.reshape(NW, NCH, K)
  dst3 = dstp.reshape(NW, NCH, K)
  hidx = jnp.concatenate([srcp, dstp + NP]).reshape(NW, 2 * NCH, K)
  x_pad = jnp.pad(x, ((0, NP - N), (0, 0)))

  hist = _hist(hidx)
  ho0, ho1 = hist[0, :NP, :], hist[1, :NP, :]
  hi0, hi1 = hist[0, NP:, :], hist[1, NP:, :]

  xs = _tc_scale(ho0, ho1, x_pad)
  parts = _agg128(xs, src3, dst3)
  gs = _tc_dense(parts[0], parts[1], hi0, hi1, ho0, ho1,
                 W1, b1.reshape(1, D), W2, b2.reshape(1, C))
  parts2 = _agg16(gs, src3, dst3)
  return _tc_read(parts2[0], parts2[1], hi0, hi1, b2.reshape(1, C))


# speedup vs baseline: 18.7601x; 1.3889x over previous
import jax
import jax.numpy as jnp
from jax import lax
from jax.experimental import pallas as pl
from jax.experimental.pallas import tpu as pltpu
from jax.experimental.pallas import tpu_sc as plsc

N = 10000
NP = 10240
E = 320000
D = 128
C = 16
NC, NS, L = 2, 16, 16
NW = NC * NS
ET = NP
EP = NW * ET
K = 128
NCH = ET // K
ROWS_PT = NP // NS
HW = 16

_MESH = plsc.VectorSubcoreMesh(
    core_axis_name="c", subcore_axis_name="s", num_cores=NC, num_subcores=NS)


def _fill_rows(buf, value, rows, width):
  vec = jnp.full((L,), value, jnp.float32)

  def body(i, carry):
    for k in range(width // L):
      buf[i, pl.ds(k * L, L)] = vec
    return carry

  lax.fori_loop(0, rows, body, 0)


def _hist_body(src_hbm, dst_hbm, hist_hbm, idx_v, ones_v, lo_sh, hi_sh,
               s0, s1, s2, s3):
  cidx = lax.axis_index("c")
  sidx = lax.axis_index("s")
  wid = cidx * NS + sidx
  sems = [s0, s1, s2, s3]
  ndepth = len(sems)

  pltpu.sync_copy(src_hbm.at[wid], idx_v.at[pl.ds(0, NCH)])
  pltpu.sync_copy(dst_hbm.at[wid], idx_v.at[pl.ds(NCH, NCH)])

  _fill_rows(ones_v, 0.0, K, HW)
  zbase = sidx * ROWS_PT
  for acc in (lo_sh, hi_sh):
    for r in range(ROWS_PT // K):
      pltpu.sync_copy(ones_v, acc.at[pl.ds(zbase + r * K, K)])
  _fill_rows(ones_v, 1.0, K, HW)
  plsc.subcore_barrier()

  for ph, acc in ((0, lo_sh), (1, hi_sh)):
    def loop(i, carry, acc=acc, ph=ph):
      for b in range(ndepth):
        jj = ph * NCH + i * ndepth + b

        @pl.when((i > 0) | (ph > 0))
        def _(b=b, acc=acc):
          pltpu.make_async_copy(ones_v, acc.at[idx_v.at[0]], sems[b]).wait()

        pltpu.async_copy(ones_v, acc.at[idx_v.at[jj]], sems[b], add=True)
      return carry

    lax.fori_loop(0, NCH // ndepth, loop, 0)
  for b in range(ndepth):
    pltpu.make_async_copy(ones_v, hi_sh.at[idx_v.at[0]], sems[b]).wait()
  plsc.subcore_barrier()

  pltpu.sync_copy(lo_sh.at[pl.ds(zbase, ROWS_PT)],
                  hist_hbm.at[cidx, 0, pl.ds(zbase, ROWS_PT)])
  pltpu.sync_copy(hi_sh.at[pl.ds(zbase, ROWS_PT)],
                  hist_hbm.at[cidx, 1, pl.ds(zbase, ROWS_PT)])


_hist = pl.kernel(
    _hist_body,
    out_type=jax.ShapeDtypeStruct((NC, 2, NP, HW), jnp.float32),
    mesh=_MESH,
    compiler_params=pltpu.CompilerParams(use_tc_tiling_on_sc=False),
    scratch_types=[
        pltpu.VMEM((2 * NCH, K), jnp.int32),
        pltpu.VMEM((K, HW), jnp.float32),
        pltpu.VMEM_SHARED((NP, HW), jnp.float32),
        pltpu.VMEM_SHARED((NP, HW), jnp.float32),
        pltpu.SemaphoreType.DMA,
        pltpu.SemaphoreType.DMA,
        pltpu.SemaphoreType.DMA,
        pltpu.SemaphoreType.DMA,
    ],
)


NHALF = 2
NCH2 = NCH // NHALF


def _make_agg(W):
  def body(table_hbm, src_hbm, dst_hbm, out_hbm,
           idx_s, idx_d, buf0, buf1, accum_sh, g0, g1, sa0, sa1):
    cidx = lax.axis_index("c")
    sidx = lax.axis_index("s")
    wid = cidx * NS + sidx
    bufs = [buf0, buf1]
    gsem = [g0, g1]
    ssem = [sa0, sa1]

    _fill_rows(buf0, 0.0, K, W)
    for r in range(ROWS_PT // K):
      pltpu.sync_copy(buf0, accum_sh.at[pl.ds(sidx * ROWS_PT + r * K, K)])
    plsc.subcore_barrier()

    for h in range(NHALF):
      pltpu.sync_copy(src_hbm.at[wid, pl.ds(h * NCH2, NCH2)], idx_s)
      pltpu.sync_copy(dst_hbm.at[wid, pl.ds(h * NCH2, NCH2)], idx_d)

      pltpu.async_copy(table_hbm.at[idx_s.at[0]], buf0, g0)

      def loop(i, carry):
        for b in range(2):
          jj = 2 * i + b
          ob = 1 - b
          pltpu.make_async_copy(table_hbm.at[idx_s.at[0]], bufs[b],
                                gsem[b]).wait()

          @pl.when(jj + 1 < NCH2)
          def _(b=b, ob=ob, jj=jj):
            @pl.when(jj >= 1)
            def _():
              pltpu.make_async_copy(bufs[ob], accum_sh.at[idx_d.at[0]],
                                    ssem[ob]).wait()
            pltpu.async_copy(table_hbm.at[idx_s.at[jj + 1]], bufs[ob],
                             gsem[ob])

          pltpu.async_copy(bufs[b], accum_sh.at[idx_d.at[jj]], ssem[b],
                           add=True)
        return carry

      lax.fori_loop(0, NCH2 // 2, loop, 0)
      pltpu.make_async_copy(bufs[0], accum_sh.at[idx_d.at[0]], ssem[0]).wait()
      pltpu.make_async_copy(bufs[1], accum_sh.at[idx_d.at[0]], ssem[1]).wait()

    plsc.subcore_barrier()

    pltpu.sync_copy(accum_sh.at[pl.ds(sidx * ROWS_PT, ROWS_PT)],
                    out_hbm.at[cidx, pl.ds(sidx * ROWS_PT, ROWS_PT)])

  return pl.kernel(
      body,
      out_type=jax.ShapeDtypeStruct((NC, NP, W), jnp.float32),
      mesh=_MESH,
      compiler_params=pltpu.CompilerParams(use_tc_tiling_on_sc=False),
      scratch_types=[
          pltpu.VMEM((NCH2, K), jnp.int32),
          pltpu.VMEM((NCH2, K), jnp.int32),
          pltpu.VMEM((K, W), jnp.float32),
          pltpu.VMEM((K, W), jnp.float32),
          pltpu.VMEM_SHARED((NP, W), jnp.float32),
          pltpu.SemaphoreType.DMA,
          pltpu.SemaphoreType.DMA,
          pltpu.SemaphoreType.DMA,
          pltpu.SemaphoreType.DMA,
      ],
  )


_agg128 = _make_agg(D)
_agg16 = _make_agg(C)


KB = 1024


def _norm(a_ref, b_ref):
  deg = a_ref[0, 0, :, 0:1] + b_ref[0, 0, :, 0:1]
  return jnp.where(deg > 0, lax.rsqrt(jnp.maximum(deg, 1.0)), 0.0)


def _hspec(which):
  return [
      pl.BlockSpec((1, 1, KB, HW), lambda i, c=c, w=which: (c, w, i, 0))
      for c in range(NC)
  ]


def _tc_scale_body(ho0, ho1, x_ref, xs_ref):
  xs_ref[...] = x_ref[...] * _norm(ho0, ho1)


_tc_scale = pl.pallas_call(
    _tc_scale_body,
    grid=(NP // KB,),
    in_specs=_hspec(0) + [pl.BlockSpec((KB, D), lambda i: (i, 0))],
    out_specs=pl.BlockSpec((KB, D), lambda i: (i, 0)),
    out_shape=jax.ShapeDtypeStruct((NP, D), jnp.float32),
)


def _tc_dense_body(p0, p1, hi0, hi1, ho0, ho1, w1, bb1, w2, bb2, out):
  a = p0[...] + p1[...]
  nd = _norm(hi0, hi1)
  ns = _norm(ho0, ho1)
  z = jnp.dot(a, w1[...], preferred_element_type=jnp.float32) * nd + bb1[...]
  h = jnp.maximum(z, 0.0)
  out[...] = jnp.dot(h, w2[...], preferred_element_type=jnp.float32) * ns


_tc_dense = pl.pallas_call(
    _tc_dense_body,
    grid=(NP // KB,),
    in_specs=[
        pl.BlockSpec((KB, D), lambda i: (i, 0)),
        pl.BlockSpec((KB, D), lambda i: (i, 0)),
    ] + _hspec(1) + _hspec(0) + [
        pl.BlockSpec((D, D), lambda i: (0, 0)),
        pl.BlockSpec((1, D), lambda i: (0, 0)),
        pl.BlockSpec((D, C), lambda i: (0, 0)),
        pl.BlockSpec((1, C), lambda i: (0, 0)),
    ],
    out_specs=pl.BlockSpec((KB, C), lambda i: (i, 0)),
    out_shape=jax.ShapeDtypeStruct((NP, C), jnp.float32),
)


def _tc_read_body(q0, q1, hi0, hi1, bb2, out):
  deg = hi0[0, 0, :, 0:1] + hi1[0, 0, :, 0:1]
  nd = jnp.where(deg > 0, lax.rsqrt(jnp.maximum(deg, 1.0)), 0.0)
  pre = jnp.maximum((q0[...] + q1[...]) * nd + bb2[...], 0.0)
  rows = lax.broadcasted_iota(jnp.int32, (NP, 1), 0)
  pre = jnp.where(rows < N, pre, 0.0)
  out[...] = jnp.sum(pre, axis=0, keepdims=True) * (1.0 / N)


_tc_read = pl.pallas_call(
    _tc_read_body,
    grid=(1,),
    in_specs=[
        pl.BlockSpec((NP, C), lambda i: (0, 0)),
        pl.BlockSpec((NP, C), lambda i: (0, 0)),
        pl.BlockSpec((1, 1, NP, HW), lambda i: (0, 1, 0, 0)),
        pl.BlockSpec((1, 1, NP, HW), lambda i: (1, 1, 0, 0)),
        pl.BlockSpec((1, C), lambda i: (0, 0)),
    ],
    out_specs=pl.BlockSpec((1, C), lambda i: (0, 0)),
    out_shape=jax.ShapeDtypeStruct((1, C), jnp.float32),
)


def kernel(x, edge_index, W1, b1, W2, b2):
  src = edge_index[0].astype(jnp.int32)
  dst = edge_index[1].astype(jnp.int32)
  padv = N + (jnp.arange(EP - E, dtype=jnp.int32) % (NP - N))
  src3 = jnp.concatenate([src, padv]).reshape(NW, NCH, K)
  dst3 = jnp.concatenate([dst, padv]).reshape(NW, NCH, K)
  x_pad = jnp.pad(x, ((0, NP - N), (0, 0)))

  hist = _hist(src3, dst3)
  xs = _tc_scale(hist, hist, x_pad)
  parts = _agg128(xs, src3, dst3)
  gs = _tc_dense(parts[0], parts[1], hist, hist, hist, hist,
                 W1, b1.reshape(1, D), W2, b2.reshape(1, C))
  parts2 = _agg16(gs, src3, dst3)
  return _tc_read(parts2[0], parts2[1], hist, hist, b2.reshape(1, C))


# speedup vs baseline: 20.4532x; 1.0902x over previous
import jax
import jax.numpy as jnp
from jax import lax
from jax.experimental import pallas as pl
from jax.experimental.pallas import tpu as pltpu
from jax.experimental.pallas import tpu_sc as plsc

N = 10000
NP = 10240
E = 320000
D = 128
C = 16
NC, NS, L = 2, 16, 16
NW = NC * NS
ET = NP
EP = NW * ET
K = 128
NCH = ET // K
ROWS_PT = NP // NS
HW = 16

_MESH = plsc.VectorSubcoreMesh(
    core_axis_name="c", subcore_axis_name="s", num_cores=NC, num_subcores=NS)


def _fill_rows(buf, value, rows, width):
  vec = jnp.full((L,), value, jnp.float32)

  def body(i, carry):
    for k in range(width // L):
      buf[i, pl.ds(k * L, L)] = vec
    return carry

  lax.fori_loop(0, rows, body, 0)


def _hist_body(src_hbm, dst_hbm, hist_hbm, idx_v, ones_v, lo_sh, hi_sh,
               s0, s1, s2, s3):
  cidx = lax.axis_index("c")
  sidx = lax.axis_index("s")
  wid = cidx * NS + sidx
  sems = [s0, s1, s2, s3]
  ndepth = len(sems)

  pltpu.sync_copy(src_hbm.at[wid], idx_v.at[pl.ds(0, NCH)])
  pltpu.sync_copy(dst_hbm.at[wid], idx_v.at[pl.ds(NCH, NCH)])

  _fill_rows(ones_v, 0.0, K, HW)
  zbase = sidx * ROWS_PT
  for acc in (lo_sh, hi_sh):
    for r in range(ROWS_PT // K):
      pltpu.sync_copy(ones_v, acc.at[pl.ds(zbase + r * K, K)])
  _fill_rows(ones_v, 1.0, K, HW)
  plsc.subcore_barrier()

  for ph, acc in ((0, lo_sh), (1, hi_sh)):
    def loop(i, carry, acc=acc, ph=ph):
      for b in range(ndepth):
        jj = ph * NCH + i * ndepth + b

        @pl.when((i > 0) | (ph > 0))
        def _(b=b, acc=acc):
          pltpu.make_async_copy(ones_v, acc.at[idx_v.at[0]], sems[b]).wait()

        pltpu.async_copy(ones_v, acc.at[idx_v.at[jj]], sems[b], add=True)
      return carry

    lax.fori_loop(0, NCH // ndepth, loop, 0)
  for b in range(ndepth):
    pltpu.make_async_copy(ones_v, hi_sh.at[idx_v.at[0]], sems[b]).wait()
  plsc.subcore_barrier()

  pltpu.sync_copy(lo_sh.at[pl.ds(zbase, ROWS_PT)],
                  hist_hbm.at[cidx, 0, pl.ds(zbase, ROWS_PT)])
  pltpu.sync_copy(hi_sh.at[pl.ds(zbase, ROWS_PT)],
                  hist_hbm.at[cidx, 1, pl.ds(zbase, ROWS_PT)])


_hist = pl.kernel(
    _hist_body,
    out_type=jax.ShapeDtypeStruct((NC, 2, NP, HW), jnp.float32),
    mesh=_MESH,
    compiler_params=pltpu.CompilerParams(use_tc_tiling_on_sc=False),
    scratch_types=[
        pltpu.VMEM((2 * NCH, K), jnp.int32),
        pltpu.VMEM((K, HW), jnp.float32),
        pltpu.VMEM_SHARED((NP, HW), jnp.float32),
        pltpu.VMEM_SHARED((NP, HW), jnp.float32),
        pltpu.SemaphoreType.DMA,
        pltpu.SemaphoreType.DMA,
        pltpu.SemaphoreType.DMA,
        pltpu.SemaphoreType.DMA,
    ],
)


NHALF = 2
NCH2 = NCH // NHALF


def _make_agg(W):
  def body(table_hbm, src_hbm, dst_hbm, out_hbm,
           idx_s, idx_d, buf0, buf1, accum_sh, g0, g1, sa0, sa1):
    cidx = lax.axis_index("c")
    sidx = lax.axis_index("s")
    wid = cidx * NS + sidx
    bufs = [buf0, buf1]
    gsem = [g0, g1]
    ssem = [sa0, sa1]

    _fill_rows(buf0, 0.0, K, W)
    for r in range(ROWS_PT // K):
      pltpu.sync_copy(buf0, accum_sh.at[pl.ds(sidx * ROWS_PT + r * K, K)])
    plsc.subcore_barrier()

    for h in range(NHALF):
      pltpu.sync_copy(src_hbm.at[wid, pl.ds(h * NCH2, NCH2)], idx_s)
      pltpu.sync_copy(dst_hbm.at[wid, pl.ds(h * NCH2, NCH2)], idx_d)

      pltpu.async_copy(table_hbm.at[idx_s.at[0]], buf0, g0)

      def loop(i, carry):
        for b in range(2):
          jj = 2 * i + b
          ob = 1 - b
          pltpu.make_async_copy(table_hbm.at[idx_s.at[0]], bufs[b],
                                gsem[b]).wait()

          @pl.when(jj + 1 < NCH2)
          def _(b=b, ob=ob, jj=jj):
            @pl.when(jj >= 1)
            def _():
              pltpu.make_async_copy(bufs[ob], accum_sh.at[idx_d.at[0]],
                                    ssem[ob]).wait()
            pltpu.async_copy(table_hbm.at[idx_s.at[jj + 1]], bufs[ob],
                             gsem[ob])

          pltpu.async_copy(bufs[b], accum_sh.at[idx_d.at[jj]], ssem[b],
                           add=True)
        return carry

      lax.fori_loop(0, NCH2 // 2, loop, 0)
      pltpu.make_async_copy(bufs[0], accum_sh.at[idx_d.at[0]], ssem[0]).wait()
      pltpu.make_async_copy(bufs[1], accum_sh.at[idx_d.at[0]], ssem[1]).wait()

    plsc.subcore_barrier()

    pltpu.sync_copy(accum_sh.at[pl.ds(sidx * ROWS_PT, ROWS_PT)],
                    out_hbm.at[cidx, pl.ds(sidx * ROWS_PT, ROWS_PT)])

  return pl.kernel(
      body,
      out_type=jax.ShapeDtypeStruct((NC, NP, W), jnp.float32),
      mesh=_MESH,
      compiler_params=None if W == D else
      pltpu.CompilerParams(use_tc_tiling_on_sc=False),
      scratch_types=[
          pltpu.VMEM((NCH2, K), jnp.int32),
          pltpu.VMEM((NCH2, K), jnp.int32),
          pltpu.VMEM((K, W), jnp.float32),
          pltpu.VMEM((K, W), jnp.float32),
          pltpu.VMEM_SHARED((NP, W), jnp.float32),
          pltpu.SemaphoreType.DMA,
          pltpu.SemaphoreType.DMA,
          pltpu.SemaphoreType.DMA,
          pltpu.SemaphoreType.DMA,
      ],
  )


_agg128 = _make_agg(D)
_agg16 = _make_agg(C)


KB = 1024


def _norm(a_ref, b_ref):
  deg = a_ref[0, 0, :, 0:1] + b_ref[0, 0, :, 0:1]
  return jnp.where(deg > 0, lax.rsqrt(jnp.maximum(deg, 1.0)), 0.0)


def _hspec(which):
  return [
      pl.BlockSpec((1, 1, KB, HW), lambda i, c=c, w=which: (c, w, i, 0))
      for c in range(NC)
  ]


def _tc_scale_body(ho0, ho1, x_ref, xs_ref):
  xs_ref[...] = x_ref[...] * _norm(ho0, ho1)


_tc_scale = pl.pallas_call(
    _tc_scale_body,
    grid=(NP // KB,),
    in_specs=_hspec(0) + [pl.BlockSpec((KB, D), lambda i: (i, 0))],
    out_specs=pl.BlockSpec((KB, D), lambda i: (i, 0)),
    out_shape=jax.ShapeDtypeStruct((NP, D), jnp.float32),
)


def _tc_dense_body(p0, p1, hi0, hi1, ho0, ho1, w1, bb1, w2, bb2, out):
  a = p0[0] + p1[0]
  nd = _norm(hi0, hi1)
  ns = _norm(ho0, ho1)
  z = jnp.dot(a, w1[...], preferred_element_type=jnp.float32) * nd + bb1[...]
  h = jnp.maximum(z, 0.0)
  out[...] = jnp.dot(h, w2[...], preferred_element_type=jnp.float32) * ns


_tc_dense = pl.pallas_call(
    _tc_dense_body,
    grid=(NP // KB,),
    in_specs=[
        pl.BlockSpec((1, KB, D), lambda i: (0, i, 0)),
        pl.BlockSpec((1, KB, D), lambda i: (1, i, 0)),
    ] + _hspec(1) + _hspec(0) + [
        pl.BlockSpec((D, D), lambda i: (0, 0)),
        pl.BlockSpec((1, D), lambda i: (0, 0)),
        pl.BlockSpec((D, C), lambda i: (0, 0)),
        pl.BlockSpec((1, C), lambda i: (0, 0)),
    ],
    out_specs=pl.BlockSpec((KB, C), lambda i: (i, 0)),
    out_shape=jax.ShapeDtypeStruct((NP, C), jnp.float32),
)


NPK = NP // 8


def _tc_read_body(q0, q1, hi0, hi1, bb2t, out):
  deg = hi0[0, 0] + hi1[0, 0]
  nd = jnp.where(deg > 0, lax.rsqrt(jnp.maximum(deg, 1.0)), 0.0)
  pre = jnp.maximum((q0[0] + q1[0]) * nd + bb2t[...], 0.0)
  node = (lax.broadcasted_iota(jnp.int32, (NPK, 128), 0) * 8
          + lax.broadcasted_iota(jnp.int32, (NPK, 128), 1) // HW)
  pre = jnp.where(node < N, pre, 0.0)
  s = jnp.sum(pre, axis=0, keepdims=True)
  acc = s[:, 0:C]
  for j in range(1, 8):
    acc = acc + s[:, j * C:(j + 1) * C]
  out[...] = acc * (1.0 / N)


_tc_read = pl.pallas_call(
    _tc_read_body,
    grid=(1,),
    in_specs=[
        pl.BlockSpec((1, NPK, 128), lambda i: (0, 0, 0)),
        pl.BlockSpec((1, NPK, 128), lambda i: (1, 0, 0)),
        pl.BlockSpec((1, 1, NPK, 128), lambda i: (0, 1, 0, 0)),
        pl.BlockSpec((1, 1, NPK, 128), lambda i: (1, 1, 0, 0)),
        pl.BlockSpec((1, 128), lambda i: (0, 0)),
    ],
    out_specs=pl.BlockSpec((1, C), lambda i: (0, 0)),
    out_shape=jax.ShapeDtypeStruct((1, C), jnp.float32),
)


def kernel(x, edge_index, W1, b1, W2, b2):
  src = edge_index[0].astype(jnp.int32)
  dst = edge_index[1].astype(jnp.int32)
  padv = N + (jnp.arange(EP - E, dtype=jnp.int32) % (NP - N))
  src3 = jnp.concatenate([src, padv]).reshape(NW, NCH, K)
  dst3 = jnp.concatenate([dst, padv]).reshape(NW, NCH, K)
  x_pad = jnp.pad(x, ((0, NP - N), (0, 0)))

  hist = _hist(src3, dst3)
  xs = _tc_scale(hist, hist, x_pad)
  parts = _agg128(xs, src3, dst3)
  gs = _tc_dense(parts, parts, hist, hist, hist, hist,
                 W1, b1.reshape(1, D), W2, b2.reshape(1, C))
  parts2 = _agg16(gs, src3, dst3)
  q_pk = parts2.reshape(NC, NP // 8, 128)
  h_pk = hist.reshape(NC, 2, NP // 8, 128)
  return _tc_read(q_pk, q_pk, h_pk, h_pk, jnp.tile(b2, 8).reshape(1, 128))


# speedup vs baseline: 21.6196x; 1.0570x over previous
import jax
import jax.numpy as jnp
from jax import lax
from jax.experimental import pallas as pl
from jax.experimental.pallas import tpu as pltpu
from jax.experimental.pallas import tpu_sc as plsc

N = 10000
NP = 10240
E = 320000
D = 128
C = 16
NC, NS, L = 2, 16, 16
NW = NC * NS
ET = NP
EP = NW * ET
K = 128
NCH = ET // K
ROWS_PT = NP // NS
HW = 16

_MESH = plsc.VectorSubcoreMesh(
    core_axis_name="c", subcore_axis_name="s", num_cores=NC, num_subcores=NS)


def _fill_rows(buf, value, rows, width):
  vec = jnp.full((L,), value, jnp.float32)

  def body(i, carry):
    for k in range(width // L):
      buf[i, pl.ds(k * L, L)] = vec
    return carry

  lax.fori_loop(0, rows, body, 0)


def _hist_body(src_hbm, dst_hbm, hist_hbm, idx_v, ones_v, lo_sh, hi_sh,
               s0, s1, s2, s3):
  cidx = lax.axis_index("c")
  sidx = lax.axis_index("s")
  wid = cidx * NS + sidx
  sems = [s0, s1, s2, s3]
  ndepth = len(sems)

  pltpu.sync_copy(src_hbm.at[wid], idx_v.at[pl.ds(0, NCH)])
  pltpu.sync_copy(dst_hbm.at[wid], idx_v.at[pl.ds(NCH, NCH)])

  _fill_rows(ones_v, 0.0, K, HW)
  zbase = sidx * ROWS_PT
  for acc in (lo_sh, hi_sh):
    for r in range(ROWS_PT // K):
      pltpu.sync_copy(ones_v, acc.at[pl.ds(zbase + r * K, K)])
  _fill_rows(ones_v, 1.0, K, HW)
  plsc.subcore_barrier()

  for ph, acc in ((0, lo_sh), (1, hi_sh)):
    def loop(i, carry, acc=acc, ph=ph):
      for b in range(ndepth):
        jj = ph * NCH + i * ndepth + b

        @pl.when((i > 0) | (ph > 0))
        def _(b=b, acc=acc):
          pltpu.make_async_copy(ones_v, acc.at[idx_v.at[0]], sems[b]).wait()

        pltpu.async_copy(ones_v, acc.at[idx_v.at[jj]], sems[b], add=True)
      return carry

    lax.fori_loop(0, NCH // ndepth, loop, 0)
  for b in range(ndepth):
    pltpu.make_async_copy(ones_v, hi_sh.at[idx_v.at[0]], sems[b]).wait()
  plsc.subcore_barrier()

  pltpu.sync_copy(lo_sh.at[pl.ds(zbase, ROWS_PT)],
                  hist_hbm.at[cidx, 0, pl.ds(zbase, ROWS_PT)])
  pltpu.sync_copy(hi_sh.at[pl.ds(zbase, ROWS_PT)],
                  hist_hbm.at[cidx, 1, pl.ds(zbase, ROWS_PT)])


_hist = pl.kernel(
    _hist_body,
    out_type=jax.ShapeDtypeStruct((NC, 2, NP, HW), jnp.float32),
    mesh=_MESH,
    compiler_params=pltpu.CompilerParams(use_tc_tiling_on_sc=False),
    scratch_types=[
        pltpu.VMEM((2 * NCH, K), jnp.int32),
        pltpu.VMEM((K, HW), jnp.float32),
        pltpu.VMEM_SHARED((NP, HW), jnp.float32),
        pltpu.VMEM_SHARED((NP, HW), jnp.float32),
        pltpu.SemaphoreType.DMA,
        pltpu.SemaphoreType.DMA,
        pltpu.SemaphoreType.DMA,
        pltpu.SemaphoreType.DMA,
    ],
)


NHALF = 2
NCH2 = NCH // NHALF


def _make_agg(W, nbuf):
  def body(table_hbm, src_hbm, dst_hbm, out_hbm,
           idx_s, idx_d, *rest):
    bufs = list(rest[:nbuf])
    accum_sh = rest[nbuf]
    gsem = list(rest[nbuf + 1:2 * nbuf + 1])
    ssem = list(rest[2 * nbuf + 1:3 * nbuf + 1])
    cidx = lax.axis_index("c")
    sidx = lax.axis_index("s")
    wid = cidx * NS + sidx

    _fill_rows(bufs[0], 0.0, K, W)
    for r in range(ROWS_PT // K):
      pltpu.sync_copy(bufs[0], accum_sh.at[pl.ds(sidx * ROWS_PT + r * K, K)])
    plsc.subcore_barrier()

    for h in range(NHALF):
      pltpu.sync_copy(src_hbm.at[wid, pl.ds(h * NCH2, NCH2)], idx_s)
      pltpu.sync_copy(dst_hbm.at[wid, pl.ds(h * NCH2, NCH2)], idx_d)

      def loop(i, carry, h=h):
        for b in range(nbuf):
          jj = i * nbuf + b

          @pl.when((i > 0) | (h > 0))
          def _(b=b):
            pltpu.make_async_copy(bufs[b], accum_sh.at[idx_d.at[0]],
                                  ssem[b]).wait()

          pltpu.async_copy(table_hbm.at[idx_s.at[jj]], bufs[b], gsem[b])
        for b in range(nbuf):
          pltpu.make_async_copy(table_hbm.at[idx_s.at[0]], bufs[b],
                                gsem[b]).wait()
        for b in range(nbuf):
          jj = i * nbuf + b
          pltpu.async_copy(bufs[b], accum_sh.at[idx_d.at[jj]], ssem[b],
                           add=True)
        return carry

      lax.fori_loop(0, NCH2 // nbuf, loop, 0)

    for b in range(nbuf):
      pltpu.make_async_copy(bufs[b], accum_sh.at[idx_d.at[0]], ssem[b]).wait()
    plsc.subcore_barrier()

    pltpu.sync_copy(accum_sh.at[pl.ds(sidx * ROWS_PT, ROWS_PT)],
                    out_hbm.at[cidx, pl.ds(sidx * ROWS_PT, ROWS_PT)])

  return pl.kernel(
      body,
      out_type=jax.ShapeDtypeStruct((NC, NP, W), jnp.float32),
      mesh=_MESH,
      compiler_params=None if W == D else
      pltpu.CompilerParams(use_tc_tiling_on_sc=False),
      scratch_types=[
          pltpu.VMEM((NCH2, K), jnp.int32),
          pltpu.VMEM((NCH2, K), jnp.int32),
      ] + [pltpu.VMEM((K, W), jnp.float32)] * nbuf
      + [pltpu.VMEM_SHARED((NP, W), jnp.float32)]
      + [pltpu.SemaphoreType.DMA] * (2 * nbuf),
  )


_agg128 = _make_agg(D, 2)
_agg16 = _make_agg(C, 4)


KB = 1024


def _norm(a_ref, b_ref):
  deg = a_ref[0, 0, :, 0:1] + b_ref[0, 0, :, 0:1]
  return jnp.where(deg > 0, lax.rsqrt(jnp.maximum(deg, 1.0)), 0.0)


def _hspec(which):
  return [
      pl.BlockSpec((1, 1, KB, HW), lambda i, c=c, w=which: (c, w, i, 0))
      for c in range(NC)
  ]


def _tc_scale_body(ho0, ho1, x_ref, xs_ref):
  xs_ref[...] = x_ref[...] * _norm(ho0, ho1)


_tc_scale = pl.pallas_call(
    _tc_scale_body,
    grid=(NP // KB,),
    in_specs=_hspec(0) + [pl.BlockSpec((KB, D), lambda i: (i, 0))],
    out_specs=pl.BlockSpec((KB, D), lambda i: (i, 0)),
    out_shape=jax.ShapeDtypeStruct((NP, D), jnp.float32),
)


def _tc_dense_body(p0, p1, hi0, hi1, ho0, ho1, w1, bb1, w2, bb2, out):
  a = p0[0] + p1[0]
  nd = _norm(hi0, hi1)
  ns = _norm(ho0, ho1)
  z = jnp.dot(a, w1[...], preferred_element_type=jnp.float32) * nd + bb1[...]
  h = jnp.maximum(z, 0.0)
  out[...] = jnp.dot(h, w2[...], preferred_element_type=jnp.float32) * ns


_tc_dense = pl.pallas_call(
    _tc_dense_body,
    grid=(NP // KB,),
    in_specs=[
        pl.BlockSpec((1, KB, D), lambda i: (0, i, 0)),
        pl.BlockSpec((1, KB, D), lambda i: (1, i, 0)),
    ] + _hspec(1) + _hspec(0) + [
        pl.BlockSpec((D, D), lambda i: (0, 0)),
        pl.BlockSpec((1, D), lambda i: (0, 0)),
        pl.BlockSpec((D, C), lambda i: (0, 0)),
        pl.BlockSpec((1, C), lambda i: (0, 0)),
    ],
    out_specs=pl.BlockSpec((KB, C), lambda i: (i, 0)),
    out_shape=jax.ShapeDtypeStruct((NP, C), jnp.float32),
)


NPK = NP // 8


def _tc_read_body(q0, q1, hi0, hi1, bb2t, out):
  deg = hi0[0, 0] + hi1[0, 0]
  nd = jnp.where(deg > 0, lax.rsqrt(jnp.maximum(deg, 1.0)), 0.0)
  pre = jnp.maximum((q0[0] + q1[0]) * nd + bb2t[...], 0.0)
  node = (lax.broadcasted_iota(jnp.int32, (NPK, 128), 0) * 8
          + lax.broadcasted_iota(jnp.int32, (NPK, 128), 1) // HW)
  pre = jnp.where(node < N, pre, 0.0)
  s = jnp.sum(pre, axis=0, keepdims=True)
  acc = s[:, 0:C]
  for j in range(1, 8):
    acc = acc + s[:, j * C:(j + 1) * C]
  out[...] = acc * (1.0 / N)


_tc_read = pl.pallas_call(
    _tc_read_body,
    grid=(1,),
    in_specs=[
        pl.BlockSpec((1, NPK, 128), lambda i: (0, 0, 0)),
        pl.BlockSpec((1, NPK, 128), lambda i: (1, 0, 0)),
        pl.BlockSpec((1, 1, NPK, 128), lambda i: (0, 1, 0, 0)),
        pl.BlockSpec((1, 1, NPK, 128), lambda i: (1, 1, 0, 0)),
        pl.BlockSpec((1, 128), lambda i: (0, 0)),
    ],
    out_specs=pl.BlockSpec((1, C), lambda i: (0, 0)),
    out_shape=jax.ShapeDtypeStruct((1, C), jnp.float32),
)


def kernel(x, edge_index, W1, b1, W2, b2):
  src = edge_index[0].astype(jnp.int32)
  dst = edge_index[1].astype(jnp.int32)
  padv = N + (jnp.arange(EP - E, dtype=jnp.int32) % (NP - N))
  src3 = jnp.concatenate([src, padv]).reshape(NW, NCH, K)
  dst3 = jnp.concatenate([dst, padv]).reshape(NW, NCH, K)
  x_pad = jnp.pad(x, ((0, NP - N), (0, 0)))

  hist = _hist(src3, dst3)
  xs = _tc_scale(hist, hist, x_pad)
  parts = _agg128(xs, src3, dst3)
  gs = _tc_dense(parts, parts, hist, hist, hist, hist,
                 W1, b1.reshape(1, D), W2, b2.reshape(1, C))
  parts2 = _agg16(gs, src3, dst3)
  q_pk = parts2.reshape(NC, NP // 8, 128)
  h_pk = hist.reshape(NC, 2, NP // 8, 128)
  return _tc_read(q_pk, q_pk, h_pk, h_pk, jnp.tile(b2, 8).reshape(1, 128))


# speedup vs baseline: 23.3598x; 1.0805x over previous
import jax
import jax.numpy as jnp
from jax import lax
from jax.experimental import pallas as pl
from jax.experimental.pallas import tpu as pltpu
from jax.experimental.pallas import tpu_sc as plsc

N = 10000
NP = 10240
E = 320000
D = 128
C = 16
NC, NS, L = 2, 16, 16
NW = NC * NS
ET = NP
EP = NW * ET
K = 128
NCH = ET // K
ROWS_PT = NP // NS
HW = 16

_MESH = plsc.VectorSubcoreMesh(
    core_axis_name="c", subcore_axis_name="s", num_cores=NC, num_subcores=NS)


def _fill_rows(buf, value, rows, width):
  vec = jnp.full((L,), value, jnp.float32)

  def body(i, carry):
    for k in range(width // L):
      buf[i, pl.ds(k * L, L)] = vec
    return carry

  lax.fori_loop(0, rows, body, 0)


def _hist_body(src_hbm, dst_hbm, hist_hbm, idx_v, ones_v, lo_sh, hi_sh,
               s0, s1, s2, s3):
  cidx = lax.axis_index("c")
  sidx = lax.axis_index("s")
  wid = cidx * NS + sidx
  sems = [s0, s1, s2, s3]
  ndepth = len(sems)

  pltpu.sync_copy(src_hbm.at[wid], idx_v.at[pl.ds(0, NCH)])
  pltpu.sync_copy(dst_hbm.at[wid], idx_v.at[pl.ds(NCH, NCH)])

  _fill_rows(ones_v, 0.0, K, HW)
  zbase = sidx * ROWS_PT
  for acc in (lo_sh, hi_sh):
    for r in range(ROWS_PT // K):
      pltpu.sync_copy(ones_v, acc.at[pl.ds(zbase + r * K, K)])
  _fill_rows(ones_v, 1.0, K, HW)
  plsc.subcore_barrier()

  for ph, acc in ((0, lo_sh), (1, hi_sh)):
    def loop(i, carry, acc=acc, ph=ph):
      for b in range(ndepth):
        jj = ph * NCH + i * ndepth + b

        @pl.when((i > 0) | (ph > 0))
        def _(b=b, acc=acc):
          pltpu.make_async_copy(ones_v, acc.at[idx_v.at[0]], sems[b]).wait()

        pltpu.async_copy(ones_v, acc.at[idx_v.at[jj]], sems[b], add=True)
      return carry

    lax.fori_loop(0, NCH // ndepth, loop, 0)
  for b in range(ndepth):
    pltpu.make_async_copy(ones_v, hi_sh.at[idx_v.at[0]], sems[b]).wait()
  plsc.subcore_barrier()

  pltpu.sync_copy(lo_sh.at[pl.ds(zbase, ROWS_PT)],
                  hist_hbm.at[cidx, 0, pl.ds(zbase, ROWS_PT)])
  pltpu.sync_copy(hi_sh.at[pl.ds(zbase, ROWS_PT)],
                  hist_hbm.at[cidx, 1, pl.ds(zbase, ROWS_PT)])


_hist = pl.kernel(
    _hist_body,
    out_type=jax.ShapeDtypeStruct((NC, 2, NP, HW), jnp.float32),
    mesh=_MESH,
    compiler_params=pltpu.CompilerParams(use_tc_tiling_on_sc=False),
    scratch_types=[
        pltpu.VMEM((2 * NCH, K), jnp.int32),
        pltpu.VMEM((K, HW), jnp.float32),
        pltpu.VMEM_SHARED((NP, HW), jnp.float32),
        pltpu.VMEM_SHARED((NP, HW), jnp.float32),
        pltpu.SemaphoreType.DMA,
        pltpu.SemaphoreType.DMA,
        pltpu.SemaphoreType.DMA,
        pltpu.SemaphoreType.DMA,
    ],
)


NHALF = 2


def _make_agg(W, nbuf, k, untiled):
  nch = ET // k
  nch2 = nch // NHALF

  def body(table_hbm, src_hbm, dst_hbm, out_hbm,
           idx_s, idx_d, *rest):
    bufs = list(rest[:nbuf])
    accum_sh = rest[nbuf]
    gsem = list(rest[nbuf + 1:2 * nbuf + 1])
    ssem = list(rest[2 * nbuf + 1:3 * nbuf + 1])
    cidx = lax.axis_index("c")
    sidx = lax.axis_index("s")
    wid = cidx * NS + sidx

    _fill_rows(bufs[0], 0.0, k, W)
    for r in range(ROWS_PT // k):
      pltpu.sync_copy(bufs[0], accum_sh.at[pl.ds(sidx * ROWS_PT + r * k, k)])
    plsc.subcore_barrier()

    for h in range(NHALF):
      pltpu.sync_copy(src_hbm.at[wid, pl.ds(h * nch2, nch2)], idx_s)
      pltpu.sync_copy(dst_hbm.at[wid, pl.ds(h * nch2, nch2)], idx_d)

      def loop(i, carry, h=h):
        for b in range(nbuf):
          jj = i * nbuf + b

          @pl.when((i > 0) | (h > 0))
          def _(b=b):
            pltpu.make_async_copy(bufs[b], accum_sh.at[idx_d.at[0]],
                                  ssem[b]).wait()

          pltpu.async_copy(table_hbm.at[idx_s.at[jj]], bufs[b], gsem[b])
        for b in range(nbuf):
          pltpu.make_async_copy(table_hbm.at[idx_s.at[0]], bufs[b],
                                gsem[b]).wait()
        for b in range(nbuf):
          jj = i * nbuf + b
          pltpu.async_copy(bufs[b], accum_sh.at[idx_d.at[jj]], ssem[b],
                           add=True)
        return carry

      lax.fori_loop(0, nch2 // nbuf, loop, 0)

    for b in range(nbuf):
      pltpu.make_async_copy(bufs[b], accum_sh.at[idx_d.at[0]], ssem[b]).wait()
    plsc.subcore_barrier()

    pltpu.sync_copy(accum_sh.at[pl.ds(sidx * ROWS_PT, ROWS_PT)],
                    out_hbm.at[cidx, pl.ds(sidx * ROWS_PT, ROWS_PT)])

  return pl.kernel(
      body,
      out_type=jax.ShapeDtypeStruct((NC, NP, W), jnp.float32),
      mesh=_MESH,
      compiler_params=pltpu.CompilerParams(use_tc_tiling_on_sc=False)
      if untiled else None,
      scratch_types=[
          pltpu.VMEM((nch2, k), jnp.int32),
          pltpu.VMEM((nch2, k), jnp.int32),
      ] + [pltpu.VMEM((k, W), jnp.float32)] * nbuf
      + [pltpu.VMEM_SHARED((NP, W), jnp.float32)]
      + [pltpu.SemaphoreType.DMA] * (2 * nbuf),
  )


K128 = 64
_agg128 = _make_agg(D, 4, K128, untiled=True)
_agg16 = _make_agg(C, 4, K, untiled=True)


KB = 1024


def _norm(a_ref, b_ref):
  deg = a_ref[0, 0, :, 0:1] + b_ref[0, 0, :, 0:1]
  return jnp.where(deg > 0, lax.rsqrt(jnp.maximum(deg, 1.0)), 0.0)


def _hspec(which):
  return [
      pl.BlockSpec((1, 1, KB, HW), lambda i, c=c, w=which: (c, w, i, 0))
      for c in range(NC)
  ]


def _tc_scale_body(ho0, ho1, x_ref, xs_ref):
  xs_ref[...] = x_ref[...] * _norm(ho0, ho1)


_tc_scale = pl.pallas_call(
    _tc_scale_body,
    grid=(NP // KB,),
    in_specs=_hspec(0) + [pl.BlockSpec((KB, D), lambda i: (i, 0))],
    out_specs=pl.BlockSpec((KB, D), lambda i: (i, 0)),
    out_shape=jax.ShapeDtypeStruct((NP, D), jnp.float32),
)


def _tc_dense_body(p0, p1, hi0, hi1, ho0, ho1, w1, bb1, w2, bb2, out):
  a = p0[0] + p1[0]
  nd = _norm(hi0, hi1)
  ns = _norm(ho0, ho1)
  z = jnp.dot(a, w1[...], preferred_element_type=jnp.float32) * nd + bb1[...]
  h = jnp.maximum(z, 0.0)
  out[...] = jnp.dot(h, w2[...], preferred_element_type=jnp.float32) * ns


_tc_dense = pl.pallas_call(
    _tc_dense_body,
    grid=(NP // KB,),
    in_specs=[
        pl.BlockSpec((1, KB, D), lambda i: (0, i, 0)),
        pl.BlockSpec((1, KB, D), lambda i: (1, i, 0)),
    ] + _hspec(1) + _hspec(0) + [
        pl.BlockSpec((D, D), lambda i: (0, 0)),
        pl.BlockSpec((1, D), lambda i: (0, 0)),
        pl.BlockSpec((D, C), lambda i: (0, 0)),
        pl.BlockSpec((1, C), lambda i: (0, 0)),
    ],
    out_specs=pl.BlockSpec((KB, C), lambda i: (i, 0)),
    out_shape=jax.ShapeDtypeStruct((NP, C), jnp.float32),
)


NPK = NP // 8


def _tc_read_body(q0, q1, hi0, hi1, bb2t, out):
  deg = hi0[0, 0] + hi1[0, 0]
  nd = jnp.where(deg > 0, lax.rsqrt(jnp.maximum(deg, 1.0)), 0.0)
  pre = jnp.maximum((q0[0] + q1[0]) * nd + bb2t[...], 0.0)
  node = (lax.broadcasted_iota(jnp.int32, (NPK, 128), 0) * 8
          + lax.broadcasted_iota(jnp.int32, (NPK, 128), 1) // HW)
  pre = jnp.where(node < N, pre, 0.0)
  s = jnp.sum(pre, axis=0, keepdims=True)
  acc = s[:, 0:C]
  for j in range(1, 8):
    acc = acc + s[:, j * C:(j + 1) * C]
  out[...] = acc * (1.0 / N)


_tc_read = pl.pallas_call(
    _tc_read_body,
    grid=(1,),
    in_specs=[
        pl.BlockSpec((1, NPK, 128), lambda i: (0, 0, 0)),
        pl.BlockSpec((1, NPK, 128), lambda i: (1, 0, 0)),
        pl.BlockSpec((1, 1, NPK, 128), lambda i: (0, 1, 0, 0)),
        pl.BlockSpec((1, 1, NPK, 128), lambda i: (1, 1, 0, 0)),
        pl.BlockSpec((1, 128), lambda i: (0, 0)),
    ],
    out_specs=pl.BlockSpec((1, C), lambda i: (0, 0)),
    out_shape=jax.ShapeDtypeStruct((1, C), jnp.float32),
)


def kernel(x, edge_index, W1, b1, W2, b2):
  src = edge_index[0].astype(jnp.int32)
  dst = edge_index[1].astype(jnp.int32)
  padv = N + (jnp.arange(EP - E, dtype=jnp.int32) % (NP - N))
  srcp = jnp.concatenate([src, padv])
  dstp = jnp.concatenate([dst, padv])
  src3 = srcp.reshape(NW, NCH, K)
  dst3 = dstp.reshape(NW, NCH, K)
  src3a = srcp.reshape(NW, ET // K128, K128)
  dst3a = dstp.reshape(NW, ET // K128, K128)
  x_pad = jnp.pad(x, ((0, NP - N), (0, 0)))

  hist = _hist(src3, dst3)
  xs = _tc_scale(hist, hist, x_pad)
  parts = _agg128(xs, src3a, dst3a)
  gs = _tc_dense(parts, parts, hist, hist, hist, hist,
                 W1, b1.reshape(1, D), W2, b2.reshape(1, C))
  parts2 = _agg16(gs, src3, dst3)
  q_pk = parts2.reshape(NC, NP // 8, 128)
  h_pk = hist.reshape(NC, 2, NP // 8, 128)
  return _tc_read(q_pk, q_pk, h_pk, h_pk, jnp.tile(b2, 8).reshape(1, 128))


# speedup vs baseline: 27.3755x; 1.1719x over previous
import jax
import jax.numpy as jnp
from jax import lax
from jax.experimental import pallas as pl
from jax.experimental.pallas import tpu as pltpu
from jax.experimental.pallas import tpu_sc as plsc

N = 10000
NP = 10240
E = 320000
D = 128
C = 16
NC, NS, L = 2, 16, 16
NW = NC * NS
ET = NP
EP = NW * ET
K = 128
NCH = ET // K
ROWS_PT = NP // NS
HW = 16

_MESH = plsc.VectorSubcoreMesh(
    core_axis_name="c", subcore_axis_name="s", num_cores=NC, num_subcores=NS)


def _fill_rows(buf, value, rows, width):
  lanes = L * (4 // buf.dtype.itemsize)
  vec = jnp.full((lanes,), value, buf.dtype)

  def body(i, carry):
    for k in range(width // lanes):
      buf[i, pl.ds(k * lanes, lanes)] = vec
    return carry

  lax.fori_loop(0, rows, body, 0)


def _hist_body(src_hbm, dst_hbm, hist_hbm, idx_v, ones_v, lo_sh, hi_sh,
               s0, s1, s2, s3, s4, s5, s6, s7):
  cidx = lax.axis_index("c")
  sidx = lax.axis_index("s")
  wid = cidx * NS + sidx
  sems = [s0, s1, s2, s3, s4, s5, s6, s7]
  ndepth = len(sems)

  pltpu.sync_copy(src_hbm.at[wid], idx_v.at[pl.ds(0, NCH)])
  pltpu.sync_copy(dst_hbm.at[wid], idx_v.at[pl.ds(NCH, NCH)])

  _fill_rows(ones_v, 0.0, K, HW)
  zbase = sidx * ROWS_PT
  for acc in (lo_sh, hi_sh):
    for r in range(ROWS_PT // K):
      pltpu.sync_copy(ones_v, acc.at[pl.ds(zbase + r * K, K)])
  _fill_rows(ones_v, 1.0, K, HW)
  plsc.subcore_barrier()

  for ph, acc in ((0, lo_sh), (1, hi_sh)):
    def loop(i, carry, acc=acc, ph=ph):
      for b in range(ndepth):
        jj = ph * NCH + i * ndepth + b

        @pl.when((i > 0) | (ph > 0))
        def _(b=b, acc=acc):
          pltpu.make_async_copy(ones_v, acc.at[idx_v.at[0]], sems[b]).wait()

        pltpu.async_copy(ones_v, acc.at[idx_v.at[jj]], sems[b], add=True)
      return carry

    lax.fori_loop(0, NCH // ndepth, loop, 0)
  for b in range(ndepth):
    pltpu.make_async_copy(ones_v, hi_sh.at[idx_v.at[0]], sems[b]).wait()
  plsc.subcore_barrier()

  pltpu.sync_copy(lo_sh.at[pl.ds(zbase, ROWS_PT)],
                  hist_hbm.at[cidx, 0, pl.ds(zbase, ROWS_PT)])
  pltpu.sync_copy(hi_sh.at[pl.ds(zbase, ROWS_PT)],
                  hist_hbm.at[cidx, 1, pl.ds(zbase, ROWS_PT)])


_hist = pl.kernel(
    _hist_body,
    out_type=jax.ShapeDtypeStruct((NC, 2, NP, HW), jnp.float32),
    mesh=_MESH,
    compiler_params=pltpu.CompilerParams(use_tc_tiling_on_sc=False),
    scratch_types=[
        pltpu.VMEM((2 * NCH, K), jnp.int32),
        pltpu.VMEM((K, HW), jnp.float32),
        pltpu.VMEM_SHARED((NP, HW), jnp.float32),
        pltpu.VMEM_SHARED((NP, HW), jnp.float32),
    ] + [pltpu.SemaphoreType.DMA] * 8,
)


NHALF = 2


def _make_agg(W, nbuf, k, untiled, dtype=jnp.float32):
  nch = ET // k
  nch2 = nch // NHALF

  def body(table_hbm, src_hbm, dst_hbm, out_hbm,
           idx_s, idx_d, *rest):
    bufs = list(rest[:nbuf])
    accum_sh = rest[nbuf]
    gsem = list(rest[nbuf + 1:2 * nbuf + 1])
    ssem = list(rest[2 * nbuf + 1:3 * nbuf + 1])
    cidx = lax.axis_index("c")
    sidx = lax.axis_index("s")
    wid = cidx * NS + sidx

    _fill_rows(bufs[0], 0.0, k, W)
    for r in range(ROWS_PT // k):
      pltpu.sync_copy(bufs[0], accum_sh.at[pl.ds(sidx * ROWS_PT + r * k, k)])
    plsc.subcore_barrier()

    for h in range(NHALF):
      pltpu.sync_copy(src_hbm.at[wid, pl.ds(h * nch2, nch2)], idx_s)
      pltpu.sync_copy(dst_hbm.at[wid, pl.ds(h * nch2, nch2)], idx_d)

      def loop(i, carry, h=h):
        for b in range(nbuf):
          jj = i * nbuf + b

          @pl.when((i > 0) | (h > 0))
          def _(b=b):
            pltpu.make_async_copy(bufs[b], accum_sh.at[idx_d.at[0]],
                                  ssem[b]).wait()

          pltpu.async_copy(table_hbm.at[idx_s.at[jj]], bufs[b], gsem[b])
        for b in range(nbuf):
          pltpu.make_async_copy(table_hbm.at[idx_s.at[0]], bufs[b],
                                gsem[b]).wait()
        for b in range(nbuf):
          jj = i * nbuf + b
          pltpu.async_copy(bufs[b], accum_sh.at[idx_d.at[jj]], ssem[b],
                           add=True)
        return carry

      lax.fori_loop(0, nch2 // nbuf, loop, 0)

    for b in range(nbuf):
      pltpu.make_async_copy(bufs[b], accum_sh.at[idx_d.at[0]], ssem[b]).wait()
    plsc.subcore_barrier()

    pltpu.sync_copy(accum_sh.at[pl.ds(sidx * ROWS_PT, ROWS_PT)],
                    out_hbm.at[cidx, pl.ds(sidx * ROWS_PT, ROWS_PT)])

  return pl.kernel(
      body,
      out_type=jax.ShapeDtypeStruct((NC, NP, W), dtype),
      mesh=_MESH,
      compiler_params=pltpu.CompilerParams(use_tc_tiling_on_sc=False)
      if untiled else None,
      scratch_types=[
          pltpu.VMEM((nch2, k), jnp.int32),
          pltpu.VMEM((nch2, k), jnp.int32),
      ] + [pltpu.VMEM((k, W), dtype)] * nbuf
      + [pltpu.VMEM_SHARED((NP, W), dtype)]
      + [pltpu.SemaphoreType.DMA] * (2 * nbuf),
  )


_agg128 = _make_agg(D, 6, K, untiled=True, dtype=jnp.bfloat16)
_agg16 = _make_agg(C, 4, K, untiled=True)


KB = 1024


def _norm(a_ref, b_ref):
  deg = a_ref[0, 0, :, 0:1] + b_ref[0, 0, :, 0:1]
  return jnp.where(deg > 0, lax.rsqrt(jnp.maximum(deg, 1.0)), 0.0)


def _hspec(which):
  return [
      pl.BlockSpec((1, 1, KB, HW), lambda i, c=c, w=which: (c, w, i, 0))
      for c in range(NC)
  ]


def _tc_scale_body(ho0, ho1, x_ref, xs_ref):
  xs_ref[...] = (x_ref[...] * _norm(ho0, ho1)).astype(jnp.bfloat16)


_tc_scale = pl.pallas_call(
    _tc_scale_body,
    grid=(NP // KB,),
    in_specs=_hspec(0) + [pl.BlockSpec((KB, D), lambda i: (i, 0))],
    out_specs=pl.BlockSpec((KB, D), lambda i: (i, 0)),
    out_shape=jax.ShapeDtypeStruct((NP, D), jnp.bfloat16),
)


def _tc_dense_body(p0, p1, hi0, hi1, ho0, ho1, w1, bb1, w2, bb2, out):
  a = (p0[0] + p1[0]).astype(jnp.float32)
  nd = _norm(hi0, hi1)
  ns = _norm(ho0, ho1)
  z = jnp.dot(a, w1[...], preferred_element_type=jnp.float32) * nd + bb1[...]
  h = jnp.maximum(z, 0.0)
  out[...] = jnp.dot(h, w2[...], preferred_element_type=jnp.float32) * ns


_tc_dense = pl.pallas_call(
    _tc_dense_body,
    grid=(NP // KB,),
    in_specs=[
        pl.BlockSpec((1, KB, D), lambda i: (0, i, 0)),
        pl.BlockSpec((1, KB, D), lambda i: (1, i, 0)),
    ] + _hspec(1) + _hspec(0) + [
        pl.BlockSpec((D, D), lambda i: (0, 0)),
        pl.BlockSpec((1, D), lambda i: (0, 0)),
        pl.BlockSpec((D, C), lambda i: (0, 0)),
        pl.BlockSpec((1, C), lambda i: (0, 0)),
    ],
    out_specs=pl.BlockSpec((KB, C), lambda i: (i, 0)),
    out_shape=jax.ShapeDtypeStruct((NP, C), jnp.float32),
)


NPK = NP // 8


def _tc_read_body(q0, q1, hi0, hi1, bb2t, out):
  deg = hi0[0, 0] + hi1[0, 0]
  nd = jnp.where(deg > 0, lax.rsqrt(jnp.maximum(deg, 1.0)), 0.0)
  pre = jnp.maximum((q0[0] + q1[0]) * nd + bb2t[...], 0.0)
  node = (lax.broadcasted_iota(jnp.int32, (NPK, 128), 0) * 8
          + lax.broadcasted_iota(jnp.int32, (NPK, 128), 1) // HW)
  pre = jnp.where(node < N, pre, 0.0)
  s = jnp.sum(pre, axis=0, keepdims=True)
  acc = s[:, 0:C]
  for j in range(1, 8):
    acc = acc + s[:, j * C:(j + 1) * C]
  out[...] = acc * (1.0 / N)


_tc_read = pl.pallas_call(
    _tc_read_body,
    grid=(1,),
    in_specs=[
        pl.BlockSpec((1, NPK, 128), lambda i: (0, 0, 0)),
        pl.BlockSpec((1, NPK, 128), lambda i: (1, 0, 0)),
        pl.BlockSpec((1, 1, NPK, 128), lambda i: (0, 1, 0, 0)),
        pl.BlockSpec((1, 1, NPK, 128), lambda i: (1, 1, 0, 0)),
        pl.BlockSpec((1, 128), lambda i: (0, 0)),
    ],
    out_specs=pl.BlockSpec((1, C), lambda i: (0, 0)),
    out_shape=jax.ShapeDtypeStruct((1, C), jnp.float32),
)


def kernel(x, edge_index, W1, b1, W2, b2):
  src = edge_index[0].astype(jnp.int32)
  dst = edge_index[1].astype(jnp.int32)
  padv = N + (jnp.arange(EP - E, dtype=jnp.int32) % (NP - N))
  srcp = jnp.concatenate([src, padv])
  dstp = jnp.concatenate([dst, padv])
  src3 = srcp.reshape(NW, NCH, K)
  dst3 = dstp.reshape(NW, NCH, K)
  x_pad = jnp.pad(x, ((0, NP - N), (0, 0)))

  hist = _hist(src3, dst3)
  xs = _tc_scale(hist, hist, x_pad)
  parts = _agg128(xs, src3, dst3)
  gs = _tc_dense(parts, parts, hist, hist, hist, hist,
                 W1, b1.reshape(1, D), W2, b2.reshape(1, C))
  parts2 = _agg16(gs, src3, dst3)
  q_pk = parts2.reshape(NC, NP // 8, 128)
  h_pk = hist.reshape(NC, 2, NP // 8, 128)
  return _tc_read(q_pk, q_pk, h_pk, h_pk, jnp.tile(b2, 8).reshape(1, 128))
